# baseline (speedup 1.0000x reference)
import functools

import jax
import jax.numpy as jnp
from jax import lax
from jax.experimental import pallas as pl
from jax.experimental.pallas import tpu as pltpu
from jax.experimental.pallas import tpu_sc as plsc

D = 64
DPAD = 128
NUM_PROMPTS = 4

_info = plsc.get_sparse_core_info()
NC, NS, LANES = _info.num_cores, _info.num_subcores, _info.num_lanes
NW = NC * NS

TILE = 128
NBUF = 5


def _make_gather(b_total, l_total):
    n_bb = b_total // TILE
    n_units = l_total * n_bb
    assert n_units % NW == 0
    per_w = n_units // NW
    mesh = plsc.VectorSubcoreMesh(core_axis_name="c", subcore_axis_name="s")

    @functools.partial(
        pl.kernel,
        mesh=mesh,
        out_type=jax.ShapeDtypeStruct(
            (l_total, D // 8, n_bb, 8, TILE), jnp.float32),
        compiler_params=pltpu.CompilerParams(
            needs_layout_passes=False, use_tc_tiling_on_sc=False),
        scratch_types=[
            pltpu.VMEM((NBUF, TILE), jnp.int32),
            pltpu.VMEM((NBUF, TILE, DPAD), jnp.float32),
            pltpu.VMEM((2, D // 8, 8, TILE), jnp.float32),
            pltpu.VMEM((NUM_PROMPTS, D), jnp.float32),
            pltpu.VMEM((LANES,), jnp.int32),
            pltpu.SemaphoreType.DMA((NBUF,)),
            pltpu.SemaphoreType.DMA((2,)),
        ],
    )
    def k(idxT_hbm, table_hbm, prompt_hbm, out_hbm, idx_v, rows_v, piece_v,
          prompt_v, red_v, gsem, wsem):
        wid = lax.axis_index("s") * NC + lax.axis_index("c")
        ubase = wid * per_w
        pltpu.sync_copy(prompt_hbm, prompt_v)

        zeros = jnp.zeros((LANES,), jnp.int32)
        ones = jnp.ones((LANES,), jnp.int32)
        lane = lax.iota(jnp.int32, LANES)

        def unit_lbb(u):
            uu = ubase + u
            return uu // n_bb, uu % n_bb

        def load_and_gather(u, b):
            l, bb = unit_lbb(u)
            pltpu.sync_copy(idxT_hbm.at[l, pl.ds(bb * TILE, TILE)],
                            idx_v.at[b])
            pltpu.async_copy(table_hbm.at[idx_v.at[b]], rows_v.at[b],
                             gsem.at[b])

        def drain_gather(b):
            pltpu.make_async_copy(table_hbm.at[idx_v.at[b]], rows_v.at[b],
                                  gsem.at[b]).wait()

        def or_reduce(acc):
            out = acc
            for sh in (8, 4, 2, 1):
                red_v[...] = out
                rot = plsc.load_gather(red_v, [(lane + sh) & (LANES - 1)])
                out = out | rot
            return out[0]

        def fix_tile(b):
            def acc_group(i, a):
                v = idx_v[b, pl.ds(i * LANES, LANES)]
                hit = (v >= 1) & (v <= NUM_PROMPTS)
                return a | jnp.where(hit, ones, zeros)

            acc = lax.fori_loop(0, TILE // LANES, acc_group, zeros)
            any_hit = or_reduce(acc)

            @pl.when(any_hit > 0)
            def _():
                def fix_group(i, c2):
                    v = idx_v[b, pl.ds(i * LANES, LANES)]
                    hit = (v >= 1) & (v <= NUM_PROMPTS)
                    rows = lane + i * LANES
                    pid = jnp.where(hit, v - 1, zeros)
                    for c in range(D):
                        cols = jnp.full((LANES,), c, jnp.int32)
                        vals = plsc.load_gather(prompt_v, [pid, cols])
                        plsc.store_scatter(rows_v.at[b], [rows, cols],
                                           vals, mask=hit)
                    return c2

                lax.fori_loop(0, TILE // LANES, fix_group, 0)

        def transpose_tile(b, pb):
            def col_body(d, c2):
                cols = jnp.full((LANES,), d, jnp.int32)
                for sub in range(TILE // LANES):
                    rows = lane + sub * LANES
                    vals = plsc.load_gather(rows_v.at[b], [rows, cols])
                    piece_v[pb, d // 8, d % 8,
                            pl.ds(sub * LANES, LANES)] = vals
                return c2

            lax.fori_loop(0, D, col_body, 0)

        def write_piece(pb, u):
            l, bb = unit_lbb(u)
            pltpu.async_copy(piece_v.at[pb],
                             out_hbm.at[l, pl.ds(0, D // 8), bb],
                             wsem.at[pb])

        def drain_write(pb, u):
            l, bb = unit_lbb(u)
            pltpu.make_async_copy(piece_v.at[pb],
                                  out_hbm.at[l, pl.ds(0, D // 8), bb],
                                  wsem.at[pb]).wait()

        for t in range(NBUF - 1):
            load_and_gather(t, t)

        def group_body(gq, carry):
            for t in range(NBUF):
                u = gq * NBUF + t
                pb_next = (t + NBUF - 1) % NBUF

                @pl.when(u + NBUF - 1 < per_w)
                def _(u=u, pb_next=pb_next):
                    load_and_gather(u + NBUF - 1, pb_next)

                drain_gather(t)
                fix_tile(t)

                @pl.when(u >= 2)
                def _(u=u):
                    drain_write(u % 2, u - 2)

                transpose_tile(t, u % 2)
                write_piece(u % 2, u)
            return carry

        lax.fori_loop(0, per_w // NBUF, group_body, 0)
        drain_write((per_w - 2) % 2, per_w - 2)
        drain_write((per_w - 1) % 2, per_w - 1)

    return k


def kernel(input, table, prompt_params):
    b, l = input.shape
    idxT = input.T
    table128 = jnp.pad(table, ((0, 0), (0, DPAD - D)))
    out5 = _make_gather(b, l)(idxT, table128, prompt_params)
    o = out5.transpose(2, 4, 0, 1, 3)
    return o.reshape(b, l, D)


# speedup vs baseline: 1.2372x; 1.2372x over previous
import functools

import jax
import jax.numpy as jnp
from jax import lax
from jax.experimental import pallas as pl
from jax.experimental.pallas import tpu as pltpu
from jax.experimental.pallas import tpu_sc as plsc

D = 64
DPAD = 128
NUM_PROMPTS = 4

_info = plsc.get_sparse_core_info()
NC, NS, LANES = _info.num_cores, _info.num_subcores, _info.num_lanes
NW = NC * NS

GATHER = 128
NGATHER = 1
CHUNK = GATHER * NGATHER
NBUF = 5


def _make_gather(n):
    assert n % (NW * CHUNK) == 0
    per_w = n // NW
    n_chunks = per_w // CHUNK
    assert n_chunks % NBUF == 0
    mesh = plsc.VectorSubcoreMesh(core_axis_name="c", subcore_axis_name="s")

    @functools.partial(
        pl.kernel,
        mesh=mesh,
        out_type=jax.ShapeDtypeStruct((n, DPAD), jnp.float32),
        compiler_params=pltpu.CompilerParams(
            needs_layout_passes=False, use_tc_tiling_on_sc=False),
        scratch_types=[
            pltpu.VMEM((NBUF, NGATHER, GATHER), jnp.int32),
            pltpu.VMEM((NBUF, CHUNK, D), jnp.float32),
            pltpu.VMEM((NUM_PROMPTS, D), jnp.float32),
            pltpu.VMEM((LANES,), jnp.int32),
            pltpu.SemaphoreType.DMA((NBUF,)),
            pltpu.SemaphoreType.DMA((NBUF,)),
        ],
    )
    def k(idx_hbm, table_hbm, prompt_hbm, out_hbm, idx_v, rows_v, prompt_v,
          red_v, gsem, wsem):
        wid = lax.axis_index("s") * NC + lax.axis_index("c")
        base_row = wid * (per_w // GATHER)
        base = wid * per_w
        pltpu.sync_copy(prompt_hbm, prompt_v)

        zeros = jnp.zeros((LANES,), jnp.int32)
        ones = jnp.ones((LANES,), jnp.int32)
        lane = lax.iota(jnp.int32, LANES)

        def load_and_gather(g, b):
            pltpu.sync_copy(
                idx_hbm.at[pl.ds(base_row + g * NGATHER, NGATHER)],
                idx_v.at[b])
            for j in range(NGATHER):
                pltpu.async_copy(
                    table_hbm.at[idx_v.at[b, j]],
                    rows_v.at[b, pl.ds(j * GATHER, GATHER)],
                    gsem.at[b])

        def drain_gather(b):
            for j in range(NGATHER):
                pltpu.make_async_copy(
                    table_hbm.at[idx_v.at[b, j]],
                    rows_v.at[b, pl.ds(j * GATHER, GATHER)],
                    gsem.at[b]).wait()

        def or_reduce(acc):
            out = acc
            for sh in (8, 4, 2, 1):
                red_v[...] = out
                rot = plsc.load_gather(red_v, [(lane + sh) & (LANES - 1)])
                out = out | rot
            return out[0]

        def fix_chunk(b):
            acc = zeros
            for j in range(NGATHER):
                def acc_group(i, a, j=j):
                    v = idx_v[b, j, pl.ds(i * LANES, LANES)]
                    hit = (v >= 1) & (v <= NUM_PROMPTS)
                    return a | jnp.where(hit, ones, zeros)

                acc = lax.fori_loop(0, GATHER // LANES, acc_group, acc)
            any_hit = or_reduce(acc)

            @pl.when(any_hit > 0)
            def _():
                for j in range(NGATHER):
                    def fix_group(i, c2, j=j):
                        v = idx_v[b, j, pl.ds(i * LANES, LANES)]
                        hit = (v >= 1) & (v <= NUM_PROMPTS)
                        rows = lane + (j * (GATHER // LANES) + i) * LANES
                        pid = jnp.where(hit, v - 1, zeros)
                        for c in range(D):
                            cols = jnp.full((LANES,), c, jnp.int32)
                            vals = plsc.load_gather(prompt_v, [pid, cols])
                            plsc.store_scatter(rows_v.at[b], [rows, cols],
                                               vals, mask=hit)
                        return c2

                    lax.fori_loop(0, GATHER // LANES, fix_group, 0)

        def write_chunk(b, off):
            pltpu.async_copy(
                rows_v.at[b],
                out_hbm.at[pl.ds(off, CHUNK), pl.ds(0, D)],
                wsem.at[b])

        def drain_write(b, off):
            pltpu.make_async_copy(
                rows_v.at[b],
                out_hbm.at[pl.ds(off, CHUNK), pl.ds(0, D)],
                wsem.at[b]).wait()

        for t in range(NBUF - 1):
            load_and_gather(t, t)

        def group_body(gq, carry):
            for t in range(NBUF):
                g = gq * NBUF + t
                pb = (t - 1) % NBUF

                @pl.when(g >= 1)
                def _(g=g, pb=pb):
                    drain_write(pb, base + (g - 1) * CHUNK)

                @pl.when(g + NBUF - 1 < n_chunks)
                def _(g=g, pb=pb):
                    load_and_gather(g + NBUF - 1, pb)

                drain_gather(t)
                fix_chunk(t)
                write_chunk(t, base + g * CHUNK)
            return carry

        lax.fori_loop(0, n_chunks // NBUF, group_body, 0)
        drain_write((n_chunks - 1) % NBUF, base + (n_chunks - 1) * CHUNK)

    return k


def kernel(input, table, prompt_params):
    b, l = input.shape
    n = b * l
    idx2d = input.reshape(n // GATHER, GATHER)
    out = _make_gather(n)(idx2d, table, prompt_params)
    return out[:, :D].reshape(b, l, D)


# speedup vs baseline: 1.3101x; 1.0590x over previous
import functools

import jax
import jax.numpy as jnp
from jax import lax
from jax.experimental import pallas as pl
from jax.experimental.pallas import tpu as pltpu
from jax.experimental.pallas import tpu_sc as plsc

D = 64
DPAD = 128
NUM_PROMPTS = 4

_info = plsc.get_sparse_core_info()
NC, NS, LANES = _info.num_cores, _info.num_subcores, _info.num_lanes
NW = NC * NS

GATHER = 128
NGATHER = 1
CHUNK = GATHER * NGATHER
NBUF = 5


def _make_gather(n):
    assert n % (NW * CHUNK) == 0
    per_w = n // NW
    n_chunks = per_w // CHUNK
    assert n_chunks % NBUF == 0
    mesh = plsc.VectorSubcoreMesh(core_axis_name="c", subcore_axis_name="s")

    @functools.partial(
        pl.kernel,
        mesh=mesh,
        out_type=jax.ShapeDtypeStruct((n, DPAD), jnp.float32),
        compiler_params=pltpu.CompilerParams(
            needs_layout_passes=False, use_tc_tiling_on_sc=False),
        scratch_types=[
            pltpu.VMEM((NBUF, NGATHER, GATHER), jnp.int32),
            pltpu.VMEM((NBUF, CHUNK, DPAD), jnp.float32),
            pltpu.VMEM((NUM_PROMPTS, D), jnp.float32),
            pltpu.VMEM((LANES,), jnp.int32),
            pltpu.SemaphoreType.DMA((NBUF,)),
            pltpu.SemaphoreType.DMA((NBUF,)),
        ],
    )
    def k(idx_hbm, table_hbm, prompt_hbm, out_hbm, idx_v, rows_v, prompt_v,
          red_v, gsem, wsem):
        wid = lax.axis_index("s") * NC + lax.axis_index("c")
        base_row = wid * (per_w // GATHER)
        base = wid * per_w
        pltpu.sync_copy(prompt_hbm, prompt_v)

        zeros = jnp.zeros((LANES,), jnp.int32)
        ones = jnp.ones((LANES,), jnp.int32)
        lane = lax.iota(jnp.int32, LANES)

        def load_and_gather(g, b):
            pltpu.sync_copy(
                idx_hbm.at[pl.ds(base_row + g * NGATHER, NGATHER)],
                idx_v.at[b])
            for j in range(NGATHER):
                pltpu.async_copy(
                    table_hbm.at[idx_v.at[b, j]],
                    rows_v.at[b, pl.ds(j * GATHER, GATHER)],
                    gsem.at[b])

        def drain_gather(b):
            for j in range(NGATHER):
                pltpu.make_async_copy(
                    table_hbm.at[idx_v.at[b, j]],
                    rows_v.at[b, pl.ds(j * GATHER, GATHER)],
                    gsem.at[b]).wait()

        def or_reduce(acc):
            out = acc
            for sh in (8, 4, 2, 1):
                red_v[...] = out
                rot = plsc.load_gather(red_v, [(lane + sh) & (LANES - 1)])
                out = out | rot
            return out[0]

        def fix_chunk(b):
            acc = zeros
            for j in range(NGATHER):
                def acc_group(i, a, j=j):
                    v = idx_v[b, j, pl.ds(i * LANES, LANES)]
                    hit = (v >= 1) & (v <= NUM_PROMPTS)
                    return a | jnp.where(hit, ones, zeros)

                acc = lax.fori_loop(0, GATHER // LANES, acc_group, acc)
            any_hit = or_reduce(acc)

            @pl.when(any_hit > 0)
            def _():
                for j in range(NGATHER):
                    def fix_group(i, c2, j=j):
                        v = idx_v[b, j, pl.ds(i * LANES, LANES)]
                        hit = (v >= 1) & (v <= NUM_PROMPTS)
                        rows = lane + (j * (GATHER // LANES) + i) * LANES
                        pid = jnp.where(hit, v - 1, zeros)
                        for c in range(D):
                            cols = jnp.full((LANES,), c, jnp.int32)
                            vals = plsc.load_gather(prompt_v, [pid, cols])
                            plsc.store_scatter(rows_v.at[b], [rows, cols],
                                               vals, mask=hit)
                        return c2

                    lax.fori_loop(0, GATHER // LANES, fix_group, 0)

        def write_chunk(b, off):
            pltpu.async_copy(
                rows_v.at[b],
                out_hbm.at[pl.ds(off, CHUNK)],
                wsem.at[b])

        def drain_write(b, off):
            pltpu.make_async_copy(
                rows_v.at[b],
                out_hbm.at[pl.ds(off, CHUNK)],
                wsem.at[b]).wait()

        for t in range(NBUF - 1):
            load_and_gather(t, t)

        def group_body(gq, carry):
            for t in range(NBUF):
                g = gq * NBUF + t
                pb = (t - 1) % NBUF

                @pl.when(g >= 1)
                def _(g=g, pb=pb):
                    drain_write(pb, base + (g - 1) * CHUNK)

                @pl.when(g + NBUF - 1 < n_chunks)
                def _(g=g, pb=pb):
                    load_and_gather(g + NBUF - 1, pb)

                drain_gather(t)
                fix_chunk(t)
                write_chunk(t, base + g * CHUNK)
            return carry

        lax.fori_loop(0, n_chunks // NBUF, group_body, 0)
        drain_write((n_chunks - 1) % NBUF, base + (n_chunks - 1) * CHUNK)

    return k


def kernel(input, table, prompt_params):
    b, l = input.shape
    n = b * l
    idx2d = input.reshape(n // GATHER, GATHER)
    table128 = jnp.pad(table, ((0, 0), (0, DPAD - D)))
    out = _make_gather(n)(idx2d, table128, prompt_params)
    return out[:, :D].reshape(b, l, D)
